# baseline (speedup 1.0000x reference)
import functools

import jax
import jax.numpy as jnp
from jax import lax
from jax.experimental import pallas as pl
from jax.experimental.pallas import tpu as pltpu
from jax.experimental.pallas import tpu_sc as plsc

_N = 4096
_E = 65536
_NC = 2
_NS = 16
_NW = _NC * _NS
_EPW = _E // _NW
_CHUNK = 128
_NCHUNK = _EPW // _CHUNK

_MESH = dict(core_axis_name="c", subcore_axis_name="s")
_SC_PARAMS = pltpu.CompilerParams(
    needs_layout_passes=False, use_tc_tiling_on_sc=False)
_DOT = dict(preferred_element_type=jnp.float32, precision=lax.Precision.HIGHEST)


def _deg_body(dst_hbm, ew_hbm, out_hbm, pk_hbm, dst_v, ew_v, deg_v, pk_v):
    c = lax.axis_index("c")
    s = lax.axis_index("s")
    wid = s * _NC + c
    pltpu.sync_copy(dst_hbm.at[wid], dst_v)
    pltpu.sync_copy(ew_hbm.at[wid], ew_v)

    def zero(i, carry):
        deg_v[pl.ds(pl.multiple_of(i * 16, 16), 16)] = jnp.zeros((16,), jnp.float32)
        return carry
    lax.fori_loop(0, _N // 16, zero, 0)

    def edge(i, carry):
        o = pl.multiple_of(i * 16, 16)
        dv = dst_v[pl.ds(o, 16)]
        wv = ew_v[pl.ds(o, 16)]
        plsc.addupdate_scatter(deg_v, [dv], wv)
        wb = (plsc.bitcast(wv, jnp.int32) + 0x8000) & jnp.int32(-65536)
        pk_v[pl.ds(o, 16)] = dv | wb
        return carry
    lax.fori_loop(0, _EPW // 16, edge, 0)

    pltpu.sync_copy(deg_v, out_hbm.at[wid])
    pltpu.sync_copy(pk_v, pk_hbm.at[wid])


_deg_kernel = pl.kernel(
    _deg_body,
    out_type=[jax.ShapeDtypeStruct((_NW, _N), jnp.float32),
              jax.ShapeDtypeStruct((_NW, _EPW), jnp.int32)],
    mesh=plsc.VectorSubcoreMesh(**_MESH),
    compiler_params=_SC_PARAMS,
    scratch_types=[
        pltpu.VMEM((_EPW,), jnp.int32),
        pltpu.VMEM((_EPW,), jnp.float32),
        pltpu.VMEM((_N,), jnp.float32),
        pltpu.VMEM((_EPW,), jnp.int32),
    ],
)


def _agg_body(F, table_hbm, src_hbm, pk_hbm, out_hbm,
              src_v, pk_v, tab_v, acc_v):
    nsl = F // 16
    tps = _NS // nsl
    p = _NC * tps
    epp = _E // p
    c = lax.axis_index("c")
    s = lax.axis_index("s")
    fslice = s % nsl
    part = c * tps + s // nsl

    pltpu.sync_copy(table_hbm.at[fslice], tab_v)

    def zero(i, carry):
        for u in range(8):
            acc_v[i * 8 + u, :] = jnp.zeros((16,), jnp.float32)
        return carry
    lax.fori_loop(0, _N // 8, zero, 0)

    lane = lax.iota(jnp.int32, 16)
    sch = 2048
    nch = sch // _CHUNK

    def superchunk(sc_i, carry):
        e0 = part * epp + sc_i * sch
        pltpu.sync_copy(src_hbm.at[pl.ds(e0, sch)], src_v)
        pltpu.sync_copy(pk_hbm.at[pl.ds(e0 // _CHUNK, nch)], pk_v)

        @plsc.parallel_loop(0, sch // 16, unroll=4)
        def group(g):
            o16 = pl.multiple_of(g * 16, 16)
            j = g // (_CHUNK // 16)
            o = pl.multiple_of((g % (_CHUNK // 16)) * 16, 16)
            pkv = pk_v[j, pl.ds(o, 16)]
            sv = src_v[pl.ds(o16, 16)]
            for t in range(16):
                pick = jnp.full((16,), t, jnp.int32)
                cs = pkv.at[pick].get(mode="promise_in_bounds")
                ss = sv.at[pick].get(mode="promise_in_bounds")
                da = cs & 0xFFFF
                ws = plsc.bitcast(cs & jnp.int32(-65536), jnp.float32)
                xw = plsc.load_gather(tab_v, [ss & (_N // 2 - 1), lane])
                amt = 16 - ((ss >> 11) << 4)
                row = plsc.bitcast((xw << amt) & jnp.int32(-65536),
                                   jnp.float32)
                plsc.addupdate_scatter(acc_v, [da, lane], row * ws)
        return carry
    lax.fori_loop(0, epp // sch, superchunk, 0)

    pltpu.sync_copy(acc_v, out_hbm.at[part, :, pl.ds(fslice * 16, 16)])


def _make_agg(F):
    nsl = F // 16
    scratch = [
        pltpu.VMEM((2048,), jnp.int32),
        pltpu.VMEM((2048 // _CHUNK, _CHUNK), jnp.int32),
        pltpu.VMEM((_N // 2, 16), jnp.int32),
        pltpu.VMEM((_N, 16), jnp.float32),
    ]
    return pl.kernel(
        functools.partial(_agg_body, F),
        out_type=jax.ShapeDtypeStruct((_NC * (_NS // nsl), _N, F), jnp.float32),
        mesh=plsc.VectorSubcoreMesh(**_MESH),
        compiler_params=_SC_PARAMS,
        scratch_types=scratch,
    )


_agg128 = _make_agg(128)
_agg256 = _make_agg(256)


def _pack_rows_body(lo_ref, hi_ref, o_ref):
    lo = lax.bitcast_convert_type(lo_ref[...], jnp.int32) + 0x8000
    hi = lax.bitcast_convert_type(hi_ref[...], jnp.int32) + 0x8000
    o_ref[...] = lax.shift_right_logical(lo, 16) | (hi & jnp.int32(-65536))


def _pack_rows(t):
    nsl = t.shape[0]
    blk = 512
    nb = (_N // 2) // blk
    return pl.pallas_call(
        _pack_rows_body,
        grid=(nb,),
        in_specs=[pl.BlockSpec((nsl, blk, 16), lambda i: (0, i, 0)),
                  pl.BlockSpec((nsl, blk, 16), lambda i: (0, i + nb, 0))],
        out_specs=pl.BlockSpec((nsl, blk, 16), lambda i: (0, i, 0)),
        out_shape=jax.ShapeDtypeStruct((nsl, _N // 2, 16), jnp.int32),
    )(t, t)


def _dinv_body(degp_ref, dinv_ref):
    dinv_ref[...] = lax.rsqrt(jnp.sum(degp_ref[...], axis=0) + 1.0)


def _dinv_kernel(degp):
    blk = 512
    return pl.pallas_call(
        _dinv_body,
        grid=(_N // blk,),
        in_specs=[pl.BlockSpec((_NW, blk), lambda i: (0, i))],
        out_specs=pl.BlockSpec((blk,), lambda i: (i,)),
        out_shape=jax.ShapeDtypeStruct((_N,), jnp.float32),
    )(degp)


def _tslices(x, o_ref):
    for sl in range(o_ref.shape[0]):
        o_ref[sl] = x[:, sl * 16:(sl + 1) * 16]


def _scale_body(x_ref, d_ref, o_ref):
    _tslices(x_ref[...] * d_ref[...], o_ref)


def _scale_kernel(x, dcol):
    blk = 512
    f = x.shape[1]
    return pl.pallas_call(
        _scale_body,
        grid=(_N // blk,),
        in_specs=[pl.BlockSpec((blk, f), lambda i: (i, 0)),
                  pl.BlockSpec((blk, 1), lambda i: (i, 0))],
        out_specs=pl.BlockSpec((f // 16, blk, 16), lambda i: (0, i, 0)),
        out_shape=jax.ShapeDtypeStruct((f // 16, _N, 16), jnp.float32),
    )(x, dcol)


def _layer_body(acc_ref, xp_ref, d_ref, W_ref, b_ref, o_ref):
    d = d_ref[...]
    xp = jnp.concatenate(
        [xp_ref[sl] for sl in range(xp_ref.shape[0])], axis=1)
    t = d * (jnp.sum(acc_ref[...], axis=0) + xp)
    h = jax.nn.relu(jnp.dot(t, W_ref[...], **_DOT) + b_ref[...])
    _tslices(h * d, o_ref)


def _layer_kernel(acc, xp, dcol, W, b):
    blk = 512
    fi, fo = W.shape
    return pl.pallas_call(
        _layer_body,
        grid=(_N // blk,),
        in_specs=[pl.BlockSpec((acc.shape[0], blk, fi), lambda i: (0, i, 0)),
                  pl.BlockSpec((fi // 16, blk, 16), lambda i: (0, i, 0)),
                  pl.BlockSpec((blk, 1), lambda i: (i, 0)),
                  pl.BlockSpec((fi, fo), lambda i: (0, 0)),
                  pl.BlockSpec((1, fo), lambda i: (0, 0))],
        out_specs=pl.BlockSpec((fo // 16, blk, 16), lambda i: (0, i, 0)),
        out_shape=jax.ShapeDtypeStruct((fo // 16, _N, 16), jnp.float32),
    )(acc, xp, dcol, W, b.reshape(1, fo))


def _head_body(acc_ref, xp_ref, d_ref, W_ref, b_ref, lW_ref, lb_ref, o_ref):
    xp = jnp.concatenate(
        [xp_ref[sl] for sl in range(xp_ref.shape[0])], axis=1)
    t = d_ref[...] * (jnp.sum(acc_ref[...], axis=0) + xp)
    h = jax.nn.relu(jnp.dot(t, W_ref[...], **_DOT) + b_ref[...])
    o_ref[...] = jnp.dot(h, lW_ref[...], **_DOT) + lb_ref[...]


def _head_kernel(acc, xp, dcol, W, b, lW, lb):
    blk = 512
    fi, fo = W.shape
    fh = lW.shape[1]
    return pl.pallas_call(
        _head_body,
        grid=(_N // blk,),
        in_specs=[pl.BlockSpec((acc.shape[0], blk, fi), lambda i: (0, i, 0)),
                  pl.BlockSpec((fi // 16, blk, 16), lambda i: (0, i, 0)),
                  pl.BlockSpec((blk, 1), lambda i: (i, 0)),
                  pl.BlockSpec((fi, fo), lambda i: (0, 0)),
                  pl.BlockSpec((1, fo), lambda i: (0, 0)),
                  pl.BlockSpec((fo, fh), lambda i: (0, 0)),
                  pl.BlockSpec((1, fh), lambda i: (0, 0))],
        out_specs=pl.BlockSpec((blk, fh), lambda i: (i, 0)),
        out_shape=jax.ShapeDtypeStruct((_N, fh), jnp.float32),
    )(acc, xp, dcol, W, b.reshape(1, fo), lW, lb.reshape(1, fh))


def _iota_body(o_ref):
    blk = o_ref.shape[1]
    k = lax.broadcasted_iota(jnp.int32, o_ref.shape, 1) + pl.program_id(0) * blk
    row = lax.broadcasted_iota(jnp.int32, o_ref.shape, 0)
    o_ref[...] = jnp.where(row == 0, k >> 12, k & (_N - 1))


def _edge_iota():
    blk = 1 << 20
    total = _N * _N
    return pl.pallas_call(
        _iota_body,
        grid=(total // blk,),
        out_specs=pl.BlockSpec((2, blk), lambda i: (0, i)),
        out_shape=jax.ShapeDtypeStruct((2, total), jnp.int32),
    )()


def kernel(z_, edge_index, edge_attr, W1, b1, W2, b2, linW, linb):
    new_edge_index = _edge_iota()

    src1 = edge_index[0]
    dstd = edge_index[1].reshape(_NW, _EPW)
    ewd = edge_attr.reshape(_NW, _EPW)

    degp, pk = _deg_kernel(dstd, ewd)
    dinv = _dinv_kernel(degp)
    dcol = dinv.reshape(_N, 1)
    pk2 = pk.reshape(_E // _CHUNK, _CHUNK)

    x0p = _scale_kernel(z_, dcol)
    acc1 = _agg128(_pack_rows(x0p), src1, pk2)
    x1p = _layer_kernel(acc1, x0p, dcol, W1, b1)
    acc2 = _agg256(_pack_rows(x1p), src1, pk2)
    out = _head_kernel(acc2, x1p, dcol, W2, b2, linW, linb)
    return (out, new_edge_index)


# speedup vs baseline: 1.0096x; 1.0096x over previous
import functools

import jax
import jax.numpy as jnp
from jax import lax
from jax.experimental import pallas as pl
from jax.experimental.pallas import tpu as pltpu
from jax.experimental.pallas import tpu_sc as plsc

_N = 4096
_E = 65536
_NC = 2
_NS = 16
_NW = _NC * _NS
_EPW = _E // _NW
_CHUNK = 128
_NCHUNK = _EPW // _CHUNK

_MESH = dict(core_axis_name="c", subcore_axis_name="s")
_SC_PARAMS = pltpu.CompilerParams(
    needs_layout_passes=False, use_tc_tiling_on_sc=False)
_DOT = dict(preferred_element_type=jnp.float32, precision=lax.Precision.HIGHEST)


def _deg_body(dst_hbm, ew_hbm, out_hbm, pk_hbm, dst_v, ew_v, deg_v, pk_v):
    c = lax.axis_index("c")
    s = lax.axis_index("s")
    wid = s * _NC + c
    pltpu.sync_copy(dst_hbm.at[wid], dst_v)
    pltpu.sync_copy(ew_hbm.at[wid], ew_v)

    def zero(i, carry):
        deg_v[pl.ds(pl.multiple_of(i * 16, 16), 16)] = jnp.zeros((16,), jnp.float32)
        return carry
    lax.fori_loop(0, _N // 16, zero, 0)

    def edge(i, carry):
        o = pl.multiple_of(i * 16, 16)
        dv = dst_v[pl.ds(o, 16)]
        wv = ew_v[pl.ds(o, 16)]
        plsc.addupdate_scatter(deg_v, [dv], wv)
        wb = (plsc.bitcast(wv, jnp.int32) + 0x8000) & jnp.int32(-65536)
        pk_v[pl.ds(o, 16)] = dv | wb
        return carry
    lax.fori_loop(0, _EPW // 16, edge, 0)

    pltpu.sync_copy(deg_v, out_hbm.at[wid])
    pltpu.sync_copy(pk_v, pk_hbm.at[wid])


_deg_kernel = pl.kernel(
    _deg_body,
    out_type=[jax.ShapeDtypeStruct((_NW, _N), jnp.float32),
              jax.ShapeDtypeStruct((_NW, _EPW), jnp.int32)],
    mesh=plsc.VectorSubcoreMesh(**_MESH),
    compiler_params=_SC_PARAMS,
    scratch_types=[
        pltpu.VMEM((_EPW,), jnp.int32),
        pltpu.VMEM((_EPW,), jnp.float32),
        pltpu.VMEM((_N,), jnp.float32),
        pltpu.VMEM((_EPW,), jnp.int32),
    ],
)


def _agg_body(F, table_hbm, src_hbm, pk_hbm, out_hbm,
              src_v, pk_v, tab_v, acc_v):
    nsl = F // 16
    tps = _NS // nsl
    p = _NC * tps
    epp = _E // p
    c = lax.axis_index("c")
    s = lax.axis_index("s")
    fslice = s % nsl
    part = c * tps + s // nsl

    pltpu.sync_copy(table_hbm.at[fslice], tab_v)

    def zero(i, carry):
        for u in range(8):
            acc_v[i * 8 + u, :] = jnp.zeros((16,), jnp.float32)
        return carry
    lax.fori_loop(0, _N // 8, zero, 0)

    lane = lax.iota(jnp.int32, 16)
    sch = 2048
    nch = sch // _CHUNK

    def superchunk(sc_i, carry):
        e0 = part * epp + sc_i * sch
        pltpu.sync_copy(src_hbm.at[pl.ds(e0, sch)], src_v)
        pltpu.sync_copy(pk_hbm.at[pl.ds(e0 // _CHUNK, nch)], pk_v)

        def chunk(j, carry2):
            @plsc.parallel_loop(0, _CHUNK // 16, unroll=2)
            def group(g):
                o = pl.multiple_of(g * 16, 16)
                pkv = pk_v[j, pl.ds(o, 16)]
                sv = src_v[pl.ds(pl.multiple_of(j * _CHUNK, 16) + o, 16)]
                for t in range(16):
                    pick = jnp.full((16,), t, jnp.int32)
                    cs = pkv.at[pick].get(mode="promise_in_bounds")
                    ss = sv.at[pick].get(mode="promise_in_bounds")
                    da = cs & 0xFFFF
                    ws = plsc.bitcast(cs & jnp.int32(-65536), jnp.float32)
                    xw = plsc.load_gather(tab_v, [ss & (_N // 2 - 1), lane])
                    amt = 16 - ((ss >> 11) << 4)
                    row = plsc.bitcast((xw << amt) & jnp.int32(-65536),
                                       jnp.float32)
                    plsc.addupdate_scatter(acc_v, [da, lane], row * ws)
            return carry2
        lax.fori_loop(0, nch, chunk, 0)
        return carry
    lax.fori_loop(0, epp // sch, superchunk, 0)

    pltpu.sync_copy(acc_v, out_hbm.at[part, :, pl.ds(fslice * 16, 16)])


def _make_agg(F):
    nsl = F // 16
    scratch = [
        pltpu.VMEM((2048,), jnp.int32),
        pltpu.VMEM((2048 // _CHUNK, _CHUNK), jnp.int32),
        pltpu.VMEM((_N // 2, 16), jnp.int32),
        pltpu.VMEM((_N, 16), jnp.float32),
    ]
    return pl.kernel(
        functools.partial(_agg_body, F),
        out_type=jax.ShapeDtypeStruct((_NC * (_NS // nsl), _N, F), jnp.float32),
        mesh=plsc.VectorSubcoreMesh(**_MESH),
        compiler_params=_SC_PARAMS,
        scratch_types=scratch,
    )


_agg128 = _make_agg(128)
_agg256 = _make_agg(256)


def _pack_rows_body(lo_ref, hi_ref, o_ref):
    lo = lax.bitcast_convert_type(lo_ref[...], jnp.int32) + 0x8000
    hi = lax.bitcast_convert_type(hi_ref[...], jnp.int32) + 0x8000
    o_ref[...] = lax.shift_right_logical(lo, 16) | (hi & jnp.int32(-65536))


def _pack_rows(t):
    nsl = t.shape[0]
    blk = 512
    nb = (_N // 2) // blk
    return pl.pallas_call(
        _pack_rows_body,
        grid=(nb,),
        in_specs=[pl.BlockSpec((nsl, blk, 16), lambda i: (0, i, 0)),
                  pl.BlockSpec((nsl, blk, 16), lambda i: (0, i + nb, 0))],
        out_specs=pl.BlockSpec((nsl, blk, 16), lambda i: (0, i, 0)),
        out_shape=jax.ShapeDtypeStruct((nsl, _N // 2, 16), jnp.int32),
    )(t, t)


def _dinv_body(degp_ref, dinv_ref):
    dinv_ref[...] = lax.rsqrt(jnp.sum(degp_ref[...], axis=0) + 1.0)


def _dinv_kernel(degp):
    blk = 512
    return pl.pallas_call(
        _dinv_body,
        grid=(_N // blk,),
        in_specs=[pl.BlockSpec((_NW, blk), lambda i: (0, i))],
        out_specs=pl.BlockSpec((blk,), lambda i: (i,)),
        out_shape=jax.ShapeDtypeStruct((_N,), jnp.float32),
    )(degp)


def _tslices(x, o_ref):
    for sl in range(o_ref.shape[0]):
        o_ref[sl] = x[:, sl * 16:(sl + 1) * 16]


def _scale_body(x_ref, d_ref, o_ref):
    _tslices(x_ref[...] * d_ref[...], o_ref)


def _scale_kernel(x, dcol):
    blk = 512
    f = x.shape[1]
    return pl.pallas_call(
        _scale_body,
        grid=(_N // blk,),
        in_specs=[pl.BlockSpec((blk, f), lambda i: (i, 0)),
                  pl.BlockSpec((blk, 1), lambda i: (i, 0))],
        out_specs=pl.BlockSpec((f // 16, blk, 16), lambda i: (0, i, 0)),
        out_shape=jax.ShapeDtypeStruct((f // 16, _N, 16), jnp.float32),
    )(x, dcol)


def _layer_body(acc_ref, xp_ref, d_ref, W_ref, b_ref, o_ref):
    d = d_ref[...]
    xp = jnp.concatenate(
        [xp_ref[sl] for sl in range(xp_ref.shape[0])], axis=1)
    t = d * (jnp.sum(acc_ref[...], axis=0) + xp)
    h = jax.nn.relu(jnp.dot(t, W_ref[...], **_DOT) + b_ref[...])
    _tslices(h * d, o_ref)


def _layer_kernel(acc, xp, dcol, W, b):
    blk = 512
    fi, fo = W.shape
    return pl.pallas_call(
        _layer_body,
        grid=(_N // blk,),
        in_specs=[pl.BlockSpec((acc.shape[0], blk, fi), lambda i: (0, i, 0)),
                  pl.BlockSpec((fi // 16, blk, 16), lambda i: (0, i, 0)),
                  pl.BlockSpec((blk, 1), lambda i: (i, 0)),
                  pl.BlockSpec((fi, fo), lambda i: (0, 0)),
                  pl.BlockSpec((1, fo), lambda i: (0, 0))],
        out_specs=pl.BlockSpec((fo // 16, blk, 16), lambda i: (0, i, 0)),
        out_shape=jax.ShapeDtypeStruct((fo // 16, _N, 16), jnp.float32),
    )(acc, xp, dcol, W, b.reshape(1, fo))


def _head_body(acc_ref, xp_ref, d_ref, W_ref, b_ref, lW_ref, lb_ref, o_ref):
    xp = jnp.concatenate(
        [xp_ref[sl] for sl in range(xp_ref.shape[0])], axis=1)
    t = d_ref[...] * (jnp.sum(acc_ref[...], axis=0) + xp)
    h = jax.nn.relu(jnp.dot(t, W_ref[...], **_DOT) + b_ref[...])
    o_ref[...] = jnp.dot(h, lW_ref[...], **_DOT) + lb_ref[...]


def _head_kernel(acc, xp, dcol, W, b, lW, lb):
    blk = 512
    fi, fo = W.shape
    fh = lW.shape[1]
    return pl.pallas_call(
        _head_body,
        grid=(_N // blk,),
        in_specs=[pl.BlockSpec((acc.shape[0], blk, fi), lambda i: (0, i, 0)),
                  pl.BlockSpec((fi // 16, blk, 16), lambda i: (0, i, 0)),
                  pl.BlockSpec((blk, 1), lambda i: (i, 0)),
                  pl.BlockSpec((fi, fo), lambda i: (0, 0)),
                  pl.BlockSpec((1, fo), lambda i: (0, 0)),
                  pl.BlockSpec((fo, fh), lambda i: (0, 0)),
                  pl.BlockSpec((1, fh), lambda i: (0, 0))],
        out_specs=pl.BlockSpec((blk, fh), lambda i: (i, 0)),
        out_shape=jax.ShapeDtypeStruct((_N, fh), jnp.float32),
    )(acc, xp, dcol, W, b.reshape(1, fo), lW, lb.reshape(1, fh))


def _iota_body(o_ref):
    blk = o_ref.shape[1]
    k = lax.broadcasted_iota(jnp.int32, o_ref.shape, 1) + pl.program_id(0) * blk
    row = lax.broadcasted_iota(jnp.int32, o_ref.shape, 0)
    o_ref[...] = jnp.where(row == 0, k >> 12, k & (_N - 1))


def _edge_iota():
    blk = 1 << 20
    total = _N * _N
    return pl.pallas_call(
        _iota_body,
        grid=(total // blk,),
        out_specs=pl.BlockSpec((2, blk), lambda i: (0, i)),
        out_shape=jax.ShapeDtypeStruct((2, total), jnp.int32),
    )()


def kernel(z_, edge_index, edge_attr, W1, b1, W2, b2, linW, linb):
    new_edge_index = _edge_iota()

    src1 = edge_index[0]
    dstd = edge_index[1].reshape(_NW, _EPW)
    ewd = edge_attr.reshape(_NW, _EPW)

    degp, pk = _deg_kernel(dstd, ewd)
    dinv = _dinv_kernel(degp)
    dcol = dinv.reshape(_N, 1)
    pk2 = pk.reshape(_E // _CHUNK, _CHUNK)

    x0p = _scale_kernel(z_, dcol)
    acc1 = _agg128(_pack_rows(x0p), src1, pk2)
    x1p = _layer_kernel(acc1, x0p, dcol, W1, b1)
    acc2 = _agg256(_pack_rows(x1p), src1, pk2)
    out = _head_kernel(acc2, x1p, dcol, W2, b2, linW, linb)
    return (out, new_edge_index)
